# baseline (speedup 1.0000x reference)
import functools

import jax
import jax.numpy as jnp
from jax import lax
from jax.experimental import pallas as pl
from jax.experimental.pallas import tpu as pltpu
from jax.experimental.pallas import tpu_sc as plsc

_VOCAB = 1_000_000
_EMBED = 32
_BATCH = 16384
_FIELDS = 26

_info = plsc.get_sparse_core_info()
_NC = _info.num_cores
_NS = _info.num_subcores
_NW = _NC * _NS
_BPW = _BATCH // _NW
_L = 16

_mesh = plsc.VectorSubcoreMesh(core_axis_name="c", subcore_axis_name="s")


@functools.partial(
    pl.kernel,
    mesh=_mesh,
    out_type=jax.ShapeDtypeStruct((_FIELDS, _EMBED, _BATCH), jnp.float32),
    scratch_types=[
        pltpu.VMEM((_FIELDS, _BPW), jnp.int32),
        pltpu.VMEM((_BPW, _EMBED), jnp.float32),
        pltpu.VMEM((_BPW, _EMBED), jnp.float32),
        pltpu.VMEM((_EMBED, _BPW), jnp.float32),
        pltpu.VMEM((_EMBED, _BPW), jnp.float32),
        pltpu.SemaphoreType.DMA,
        pltpu.SemaphoreType.DMA,
        pltpu.SemaphoreType.DMA,
        pltpu.SemaphoreType.DMA,
    ],
    compiler_params=pltpu.CompilerParams(
        use_tc_tiling_on_sc=False, needs_layout_passes=False),
)
def _gather_all(table_hbm, idx_hbm, out_hbm, idx_v, r0, r1, t0buf, t1buf,
                gsem0, gsem1, wsem0, wsem1):
    wid = lax.axis_index("s") * _NC + lax.axis_index("c")
    b0 = wid * _BPW
    rbufs = (r0, r1)
    tbufs = (t0buf, t1buf)
    gsems = (gsem0, gsem1)
    wsems = (wsem0, wsem1)

    pltpu.sync_copy(idx_hbm.at[:, pl.ds(b0, _BPW)], idx_v)

    def gstart(f):
        return pltpu.async_copy(
            table_hbm.at[idx_v.at[f]], rbufs[f % 2], gsems[f % 2])

    def wstart(f):
        return pltpu.async_copy(
            tbufs[f % 2], out_hbm.at[f, :, pl.ds(b0, _BPW)], wsems[f % 2])

    iota = lax.iota(jnp.int32, _L)

    def transpose_chunk(r, t):
        def body(i, carry):
            rid = i * _L + iota
            for e in range(_EMBED):
                ce = jnp.full((_L,), e, jnp.int32)
                vals = plsc.load_gather(r, [rid, ce])
                t[e, pl.ds(i * _L, _L)] = vals
            return carry
        lax.fori_loop(0, _BPW // _L, body, 0)

    g = [None] * _FIELDS
    w = [None] * _FIELDS
    g[0] = gstart(0)
    for f in range(_FIELDS):
        if f + 1 < _FIELDS:
            g[f + 1] = gstart(f + 1)
        g[f].wait()
        if f >= 2:
            w[f - 2].wait()
        transpose_chunk(rbufs[f % 2], tbufs[f % 2])
        w[f] = wstart(f)
    w[_FIELDS - 2].wait()
    w[_FIELDS - 1].wait()


def kernel(tokenIndex, weights):
    idx_t = tokenIndex.T.astype(jnp.int32)
    out_t = _gather_all(weights, idx_t)
    return out_t.transpose(2, 0, 1)


# speedup vs baseline: 1.1704x; 1.1704x over previous
import functools

import jax
import jax.numpy as jnp
from jax import lax
from jax.experimental import pallas as pl
from jax.experimental.pallas import tpu as pltpu
from jax.experimental.pallas import tpu_sc as plsc

_VOCAB = 1_000_000
_EMBED = 32
_BATCH = 16384
_FIELDS = 26

_info = plsc.get_sparse_core_info()
_NC = _info.num_cores
_NS = _info.num_subcores
_NW = _NC * _NS
_BPW = _BATCH // _NW
_L = 16

_mesh = plsc.VectorSubcoreMesh(core_axis_name="c", subcore_axis_name="s")


@functools.partial(
    pl.kernel,
    mesh=_mesh,
    out_type=jax.ShapeDtypeStruct((_FIELDS, _EMBED, _BATCH), jnp.float32),
    scratch_types=[
        pltpu.VMEM((_FIELDS, _BPW), jnp.int32),
        pltpu.VMEM((_BPW, _EMBED), jnp.float32),
        pltpu.VMEM((_BPW, _EMBED), jnp.float32),
        pltpu.VMEM((_EMBED, _BPW), jnp.float32),
        pltpu.VMEM((_EMBED, _BPW), jnp.float32),
        pltpu.SemaphoreType.DMA,
        pltpu.SemaphoreType.DMA,
        pltpu.SemaphoreType.DMA,
        pltpu.SemaphoreType.DMA,
    ],
    compiler_params=pltpu.CompilerParams(
        use_tc_tiling_on_sc=False, needs_layout_passes=False),
)
def _gather_all(table_hbm, idx_hbm, out_hbm, idx_v, r0, r1, t0buf, t1buf,
                gsem0, gsem1, wsem0, wsem1):
    wid = lax.axis_index("s") * _NC + lax.axis_index("c")
    b0 = wid * _BPW
    rbufs = (r0, r1)
    tbufs = (t0buf, t1buf)
    gsems = (gsem0, gsem1)
    wsems = (wsem0, wsem1)

    pltpu.sync_copy(idx_hbm.at[:, pl.ds(b0, _BPW)], idx_v)

    def gdesc(f, p):
        return pltpu.make_async_copy(
            table_hbm.at[idx_v.at[f]], rbufs[p], gsems[p])

    def wdesc(f, p):
        return pltpu.make_async_copy(
            tbufs[p], out_hbm.at[f, :, pl.ds(b0, _BPW)], wsems[p])

    iota = lax.iota(jnp.int32, _L)
    ces = [jnp.full((_L,), e, jnp.int32) for e in range(_EMBED)]

    def transpose_chunk(r, t):
        @plsc.parallel_loop(0, _BPW // _L, unroll=2)
        def _(i):
            rid = i * _L + iota
            for e in range(_EMBED):
                vals = plsc.load_gather(r, [rid, ces[e]])
                t[e, pl.ds(i * _L, _L)] = vals

    def step(f, p):
        fn = jnp.minimum(f + 1, _FIELDS - 1)

        @pl.when(f + 1 < _FIELDS)
        def _():
            gdesc(fn, 1 - p).start()

        gdesc(f, p).wait()

        @pl.when(f >= 2)
        def _():
            wdesc(f - 2, p).wait()

        transpose_chunk(rbufs[p], tbufs[p])
        wdesc(f, p).start()

    gdesc(0, 0).start()

    def body(k, carry):
        step(2 * k, 0)
        step(2 * k + 1, 1)
        return carry

    lax.fori_loop(0, _FIELDS // 2, body, 0)
    wdesc(_FIELDS - 2, 0).wait()
    wdesc(_FIELDS - 1, 1).wait()


def kernel(tokenIndex, weights):
    idx_t = tokenIndex.T.astype(jnp.int32)
    out_t = _gather_all(weights, idx_t)
    return out_t.transpose(2, 0, 1)


# speedup vs baseline: 1.4780x; 1.2628x over previous
import functools

import jax
import jax.numpy as jnp
from jax import lax
from jax.experimental import pallas as pl
from jax.experimental.pallas import tpu as pltpu
from jax.experimental.pallas import tpu_sc as plsc

_VOCAB = 1_000_000
_EMBED = 32
_BATCH = 16384
_FIELDS = 26

_info = plsc.get_sparse_core_info()
_NC = _info.num_cores
_NS = _info.num_subcores
_NW = _NC * _NS
_BPW = _BATCH // _NW
_L = 16

_mesh = plsc.VectorSubcoreMesh(core_axis_name="c", subcore_axis_name="s")


@functools.partial(
    pl.kernel,
    mesh=_mesh,
    out_type=jax.ShapeDtypeStruct((_FIELDS, _EMBED, _BATCH), jnp.float32),
    scratch_types=[
        pltpu.VMEM((_FIELDS, _BPW), jnp.int32),
        pltpu.VMEM((_BPW, _EMBED), jnp.float32),
        pltpu.VMEM((_BPW, _EMBED), jnp.float32),
        pltpu.VMEM((_EMBED, _BPW + 1), jnp.float32),
        pltpu.VMEM((_EMBED, _BPW + 1), jnp.float32),
        pltpu.SemaphoreType.DMA,
        pltpu.SemaphoreType.DMA,
        pltpu.SemaphoreType.DMA,
        pltpu.SemaphoreType.DMA,
    ],
    compiler_params=pltpu.CompilerParams(
        use_tc_tiling_on_sc=False, needs_layout_passes=False),
)
def _gather_all(table_hbm, idx_hbm, out_hbm, idx_v, r0, r1, t0buf, t1buf,
                gsem0, gsem1, wsem0, wsem1):
    wid = lax.axis_index("s") * _NC + lax.axis_index("c")
    b0 = wid * _BPW
    rbufs = (r0, r1)
    tbufs = (t0buf, t1buf)
    gsems = (gsem0, gsem1)
    wsems = (wsem0, wsem1)

    pltpu.sync_copy(idx_hbm.at[:, pl.ds(b0, _BPW)], idx_v)

    def gdesc(f, p):
        return pltpu.make_async_copy(
            table_hbm.at[idx_v.at[f]], rbufs[p], gsems[p])

    def wdesc(f, p):
        return pltpu.make_async_copy(
            tbufs[p].at[:, pl.ds(0, _BPW)],
            out_hbm.at[f, :, pl.ds(b0, _BPW)], wsems[p])

    iota = lax.iota(jnp.int32, _L)
    rows0 = iota
    rows1 = iota + _L

    def transpose_chunk(r, t):
        @plsc.parallel_loop(0, _BPW, unroll=4)
        def _(i):
            ci = jnp.full((_L,), i, jnp.int32)
            v0 = r[i, pl.ds(0, _L)]
            v1 = r[i, pl.ds(_L, _L)]
            plsc.store_scatter(t, [rows0, ci], v0)
            plsc.store_scatter(t, [rows1, ci], v1)

    def step(f, p):
        fn = jnp.minimum(f + 1, _FIELDS - 1)

        @pl.when(f + 1 < _FIELDS)
        def _():
            gdesc(fn, 1 - p).start()

        gdesc(f, p).wait()

        @pl.when(f >= 2)
        def _():
            wdesc(f - 2, p).wait()

        transpose_chunk(rbufs[p], tbufs[p])
        wdesc(f, p).start()

    gdesc(0, 0).start()

    def body(k, carry):
        step(2 * k, 0)
        step(2 * k + 1, 1)
        return carry

    lax.fori_loop(0, _FIELDS // 2, body, 0)
    wdesc(_FIELDS - 2, 0).wait()
    wdesc(_FIELDS - 1, 1).wait()


def kernel(tokenIndex, weights):
    idx_t = tokenIndex.T.astype(jnp.int32)
    out_t = _gather_all(weights, idx_t)
    return out_t.transpose(2, 0, 1)
